# speedup vs baseline: 3.4015x; 3.4015x over previous
import functools

import jax
import jax.numpy as jnp
from jax import lax
from jax.experimental import pallas as pl
from jax.experimental.pallas import tpu as pltpu
from jax.experimental.pallas import tpu_sc as plsc

M, K, D, N = 10000, 32, 128, 10000
L = 16
NC, NS = 2, 16
NW = NC * NS
R = 4
CHUNK_IDX = R * K
NCHUNK = M // R
GPW = -(-NCHUNK // NW)

_mesh = plsc.VectorSubcoreMesh(
    core_axis_name="c", subcore_axis_name="s", num_cores=NC, num_subcores=NS
)


@functools.partial(
    pl.kernel,
    out_type=jax.ShapeDtypeStruct((M, D), jnp.float32),
    mesh=_mesh,
    scratch_types=[
        pltpu.VMEM((CHUNK_IDX,), jnp.int32),
        pltpu.VMEM((CHUNK_IDX, D), jnp.float32),
        pltpu.VMEM((R, D), jnp.float32),
        pltpu.SemaphoreType.DMA,
    ],
)
def _maxpool_sc(feats_hbm, idx_hbm, out_hbm, idx_v, rows_v, out_v, sem):
    wid = lax.axis_index("s") * NC + lax.axis_index("c")

    @pl.loop(0, GPW)
    def _chunk_loop(g):
        chunk = g * NW + wid

        @pl.when(chunk < NCHUNK)
        def _():
            pltpu.sync_copy(
                idx_hbm.at[pl.ds(chunk * CHUNK_IDX, CHUNK_IDX)], idx_v
            )
            pltpu.async_copy(feats_hbm.at[idx_v], rows_v, sem).wait()

            def k_body(k, accs):
                return tuple(
                    jnp.maximum(
                        accs[r * (D // L) + c],
                        rows_v[r * K + k, pl.ds(c * L, L)],
                    )
                    for r in range(R)
                    for c in range(D // L)
                )

            init = tuple(
                rows_v[r * K, pl.ds(c * L, L)]
                for r in range(R)
                for c in range(D // L)
            )
            accs = lax.fori_loop(1, K, k_body, init)
            for r in range(R):
                for c in range(D // L):
                    out_v[r, pl.ds(c * L, L)] = accs[r * (D // L) + c]
            pltpu.sync_copy(out_v, out_hbm.at[pl.ds(chunk * R, R)])


def kernel(s_feats, neighbor_indices):
    idx = neighbor_indices.astype(jnp.int32).reshape(M * K)
    return _maxpool_sc(s_feats, idx)
